# Initial kernel scaffold; baseline (speedup 1.0000x reference)
import jax
import jax.numpy as jnp
from jax.experimental import pallas as pl


def kernel(boxes, scores):
    raise NotImplementedError("write your pallas kernel here")


# speedup vs baseline: 4.0902x; 4.0902x over previous
import jax
import jax.numpy as jnp
from jax.experimental import pallas as pl
from jax.experimental.pallas import tpu as pltpu

_N = 5000
_K = 80
_M = 1000
_TOPK = 100
_SCORE_THRESH = 0.05
_NMS_THRESH = 0.5
_IMG_W = 1024.0
_CLS_OFFSET = 4096.0
_CHUNK = 1000


def _mask_kernel(scores_ref, out_ref):
    s = scores_ref[:, :_K]
    out_ref[:, :] = jnp.where(s > _SCORE_THRESH, s, -1.0)


def _nms_kernel(boxes_ref, boxest_ref, vals_row_ref, vals_col_ref,
                idx_row_ref, idx_col_ref, out_ref, cls_ref, ind_ref, s_ref):
    f32 = jnp.float32

    b = jnp.clip(boxes_ref[:, :], 0.0, _IMG_W)
    bt = jnp.clip(boxest_ref[:, :], 0.0, _IMG_W)

    idx_col = idx_col_ref[:, :]
    idx_row = idx_row_ref[:, :]
    n_col = idx_col // _K
    c_col = idx_col - n_col * _K
    n_row = idx_row // _K
    c_row = idx_row - n_row * _K

    cand = jnp.zeros((_M, 4), dtype=f32)
    candt = jnp.zeros((4, _M), dtype=f32)
    for c in range(0, _N, _CHUNK):
        lane = jax.lax.broadcasted_iota(jnp.int32, (_M, _CHUNK), 1) + c
        oh = (lane == n_col).astype(f32)
        cand = cand + jnp.dot(oh, b[c:c + _CHUNK, :],
                              preferred_element_type=f32)
        sub = jax.lax.broadcasted_iota(jnp.int32, (_CHUNK, _M), 0) + c
        oht = (sub == n_row).astype(f32)
        candt = candt + jnp.dot(bt[:, c:c + _CHUNK], oht,
                                preferred_element_type=f32)

    off_col = c_col.astype(f32) * _CLS_OFFSET
    off_row = c_row.astype(f32) * _CLS_OFFSET
    sx1c = cand[:, 0:1] + off_col
    sy1c = cand[:, 1:2] + off_col
    sx2c = cand[:, 2:3] + off_col
    sy2c = cand[:, 3:4] + off_col
    x1r = candt[0:1, :]
    y1r = candt[1:2, :]
    x2r = candt[2:3, :]
    y2r = candt[3:4, :]
    sx1r = x1r + off_row
    sy1r = y1r + off_row
    sx2r = x2r + off_row
    sy2r = y2r + off_row

    area_c = (sx2c - sx1c) * (sy2c - sy1c)
    area_r = (sx2r - sx1r) * (sy2r - sy1r)

    xx1 = jnp.maximum(sx1c, sx1r)
    yy1 = jnp.maximum(sy1c, sy1r)
    xx2 = jnp.minimum(sx2c, sx2r)
    yy2 = jnp.minimum(sy2c, sy2r)
    iw = jnp.clip(xx2 - xx1, 0.0)
    ih = jnp.clip(yy2 - yy1, 0.0)
    inter = iw * ih
    iou = inter / (area_c + area_r - inter + 1e-9)

    vals_row = vals_row_ref[:, :]
    vals_col = vals_col_ref[:, :]
    ri = jax.lax.broadcasted_iota(jnp.int32, (_M, _M), 0)
    ci = jax.lax.broadcasted_iota(jnp.int32, (_M, _M), 1)
    s_ref[:, :] = jnp.where(
        (iou > _NMS_THRESH) & (ci > ri) & (vals_col > 0.0), 1.0, 0.0)

    lane_i = jax.lax.broadcasted_iota(jnp.int32, (1, _M), 1)

    def nms_body(i, keep):
        row = s_ref[pl.ds(i, 1), :]
        keep_i = jnp.sum(keep * (lane_i == i).astype(f32))
        return keep * (1.0 - row * keep_i)

    keep = jax.lax.fori_loop(0, _M, nms_body,
                             jnp.ones((1, _M), dtype=f32))

    kept = jnp.where((keep > 0.5) & (vals_row > 0.0), vals_row, -1.0)

    lane_f = lane_i.astype(f32)
    c_rowf = c_row.astype(f32)
    n_rowf = n_row.astype(f32)

    def top_body(t, cur):
        m = jnp.max(cur)
        jf = jnp.min(jnp.where(cur == m, lane_f, 1e9))
        oh = (lane_f == jf).astype(f32)
        out_ref[pl.ds(t, 1), pl.ds(0, 1)] = jnp.sum(x1r * oh).reshape(1, 1)
        out_ref[pl.ds(t, 1), pl.ds(1, 1)] = jnp.sum(y1r * oh).reshape(1, 1)
        out_ref[pl.ds(t, 1), pl.ds(2, 1)] = jnp.sum(x2r * oh).reshape(1, 1)
        out_ref[pl.ds(t, 1), pl.ds(3, 1)] = jnp.sum(y2r * oh).reshape(1, 1)
        out_ref[pl.ds(t, 1), pl.ds(4, 1)] = m.reshape(1, 1)
        cls_ref[pl.ds(t, 1), :] = jnp.sum(c_rowf * oh).astype(jnp.int32).reshape(1, 1)
        ind_ref[pl.ds(t, 1), :] = jnp.sum(n_rowf * oh).astype(jnp.int32).reshape(1, 1)
        return jnp.where(oh > 0.5, -3.0, cur)

    jax.lax.fori_loop(0, _TOPK, top_body, kept)


@jax.jit
def kernel(boxes, scores):
    masked = pl.pallas_call(
        _mask_kernel,
        out_shape=jax.ShapeDtypeStruct((_N, _K), jnp.float32),
    )(scores)
    vals, idx = jax.lax.top_k(masked.reshape(-1), _M)
    out, cls, ind = pl.pallas_call(
        _nms_kernel,
        out_shape=(
            jax.ShapeDtypeStruct((_TOPK, 5), jnp.float32),
            jax.ShapeDtypeStruct((_TOPK, 1), jnp.int32),
            jax.ShapeDtypeStruct((_TOPK, 1), jnp.int32),
        ),
        scratch_shapes=[pltpu.VMEM((_M, _M), jnp.float32)],
    )(boxes, boxes.T, vals.reshape(1, _M), vals.reshape(_M, 1),
      idx.reshape(1, _M), idx.reshape(_M, 1))
    return out, cls.reshape(-1), ind.reshape(-1)


# speedup vs baseline: 5.3912x; 1.3181x over previous
import jax
import jax.numpy as jnp
from jax.experimental import pallas as pl
from jax.experimental.pallas import tpu as pltpu

_N = 5000
_K = 80
_M = 1000
_TOPK = 100
_SCORE_THRESH = 0.05
_NMS_THRESH = 0.5
_IMG_W = 1024.0
_CLS_OFFSET = 4096.0
_CHUNK = 1000


_NCHUNK = 256
_CSZ = 1568
_PAD = _NCHUNK * _CSZ - _N * _K


def _mask_kernel(scores_ref, out_ref):
    s = scores_ref[:, :_K]
    out_ref[:, :] = jnp.where(s > _SCORE_THRESH, s, -1.0)


def _select_kernel(d_ref, vals_ref, idx_ref, w_ref):
    f32 = jnp.float32
    sub_colf = jax.lax.broadcasted_iota(jnp.int32, (_NCHUNK, 1), 0).astype(f32)
    lane_rowf = jax.lax.broadcasted_iota(jnp.int32, (1, _CSZ), 1).astype(f32)

    def body(t, cm):
        m = jnp.max(cm)
        cf = jnp.min(jnp.where(cm == m, sub_colf, 1e9))
        c = cf.astype(jnp.int32)
        row = w_ref[pl.ds(c, 1), :]
        lf = jnp.min(jnp.where(row == m, lane_rowf, 1e9))
        row_new = jnp.where(lane_rowf == lf, -5.0, row)
        w_ref[pl.ds(c, 1), :] = row_new
        new_max = jnp.max(row_new)
        vals_ref[pl.ds(t, 1), :] = m.reshape(1, 1)
        idx_ref[pl.ds(t, 1), :] = (c * _CSZ + lf.astype(jnp.int32)).reshape(1, 1)
        return jnp.where(sub_colf == cf, new_max, cm)

    d = d_ref[:, :]
    w_ref[:, :] = d
    cm0 = jnp.max(d, axis=1, keepdims=True)
    jax.lax.fori_loop(0, _M, body, cm0)


def _nms_kernel(boxes_ref, boxest_ref, vals_row_ref, vals_col_ref,
                idx_row_ref, idx_col_ref, out_ref, cls_ref, ind_ref, s_ref):
    f32 = jnp.float32

    b = jnp.clip(boxes_ref[:, :], 0.0, _IMG_W)
    bt = jnp.clip(boxest_ref[:, :], 0.0, _IMG_W)

    idx_col = idx_col_ref[:, :]
    idx_row = idx_row_ref[:, :]
    n_col = idx_col // _K
    c_col = idx_col - n_col * _K
    n_row = idx_row // _K
    c_row = idx_row - n_row * _K

    cand = jnp.zeros((_M, 4), dtype=f32)
    candt = jnp.zeros((4, _M), dtype=f32)
    for c in range(0, _N, _CHUNK):
        lane = jax.lax.broadcasted_iota(jnp.int32, (_M, _CHUNK), 1) + c
        oh = (lane == n_col).astype(f32)
        cand = cand + jnp.dot(oh, b[c:c + _CHUNK, :],
                              preferred_element_type=f32)
        sub = jax.lax.broadcasted_iota(jnp.int32, (_CHUNK, _M), 0) + c
        oht = (sub == n_row).astype(f32)
        candt = candt + jnp.dot(bt[:, c:c + _CHUNK], oht,
                                preferred_element_type=f32)

    off_col = c_col.astype(f32) * _CLS_OFFSET
    off_row = c_row.astype(f32) * _CLS_OFFSET
    sx1c = cand[:, 0:1] + off_col
    sy1c = cand[:, 1:2] + off_col
    sx2c = cand[:, 2:3] + off_col
    sy2c = cand[:, 3:4] + off_col
    x1r = candt[0:1, :]
    y1r = candt[1:2, :]
    x2r = candt[2:3, :]
    y2r = candt[3:4, :]
    sx1r = x1r + off_row
    sy1r = y1r + off_row
    sx2r = x2r + off_row
    sy2r = y2r + off_row

    area_c = (sx2c - sx1c) * (sy2c - sy1c)
    area_r = (sx2r - sx1r) * (sy2r - sy1r)

    xx1 = jnp.maximum(sx1c, sx1r)
    yy1 = jnp.maximum(sy1c, sy1r)
    xx2 = jnp.minimum(sx2c, sx2r)
    yy2 = jnp.minimum(sy2c, sy2r)
    iw = jnp.clip(xx2 - xx1, 0.0)
    ih = jnp.clip(yy2 - yy1, 0.0)
    inter = iw * ih
    iou = inter / (area_c + area_r - inter + 1e-9)

    vals_row = vals_row_ref[:, :]
    vals_col = vals_col_ref[:, :]
    ri = jax.lax.broadcasted_iota(jnp.int32, (_M, _M), 0)
    ci = jax.lax.broadcasted_iota(jnp.int32, (_M, _M), 1)
    s_ref[:, :] = jnp.where(
        (iou > _NMS_THRESH) & (ci > ri) & (vals_col > 0.0), 1.0, 0.0)

    lane_i = jax.lax.broadcasted_iota(jnp.int32, (1, _M), 1)

    def nms_body(i, keep):
        row = s_ref[pl.ds(i, 1), :]
        keep_i = jnp.sum(keep * (lane_i == i).astype(f32))
        return keep * (1.0 - row * keep_i)

    keep = jax.lax.fori_loop(0, _M, nms_body,
                             jnp.ones((1, _M), dtype=f32))

    kept = jnp.where((keep > 0.5) & (vals_row > 0.0), vals_row, -1.0)

    lane_f = lane_i.astype(f32)
    c_rowf = c_row.astype(f32)
    n_rowf = n_row.astype(f32)

    def top_body(t, cur):
        m = jnp.max(cur)
        jf = jnp.min(jnp.where(cur == m, lane_f, 1e9))
        oh = (lane_f == jf).astype(f32)
        out_ref[pl.ds(t, 1), pl.ds(0, 1)] = jnp.sum(x1r * oh).reshape(1, 1)
        out_ref[pl.ds(t, 1), pl.ds(1, 1)] = jnp.sum(y1r * oh).reshape(1, 1)
        out_ref[pl.ds(t, 1), pl.ds(2, 1)] = jnp.sum(x2r * oh).reshape(1, 1)
        out_ref[pl.ds(t, 1), pl.ds(3, 1)] = jnp.sum(y2r * oh).reshape(1, 1)
        out_ref[pl.ds(t, 1), pl.ds(4, 1)] = m.reshape(1, 1)
        cls_ref[pl.ds(t, 1), :] = jnp.sum(c_rowf * oh).astype(jnp.int32).reshape(1, 1)
        ind_ref[pl.ds(t, 1), :] = jnp.sum(n_rowf * oh).astype(jnp.int32).reshape(1, 1)
        return jnp.where(oh > 0.5, -3.0, cur)

    jax.lax.fori_loop(0, _TOPK, top_body, kept)


@jax.jit
def kernel(boxes, scores):
    masked = pl.pallas_call(
        _mask_kernel,
        out_shape=jax.ShapeDtypeStruct((_N, _K), jnp.float32),
    )(scores)
    flat = jnp.concatenate(
        [masked.reshape(-1), jnp.full((_PAD,), -5.0, jnp.float32)])
    vals_c, idx_c = pl.pallas_call(
        _select_kernel,
        out_shape=(
            jax.ShapeDtypeStruct((_M, 1), jnp.float32),
            jax.ShapeDtypeStruct((_M, 1), jnp.int32),
        ),
        scratch_shapes=[pltpu.VMEM((_NCHUNK, _CSZ), jnp.float32)],
    )(flat.reshape(_NCHUNK, _CSZ))
    vals = vals_c.reshape(-1)
    idx = idx_c.reshape(-1)
    out, cls, ind = pl.pallas_call(
        _nms_kernel,
        out_shape=(
            jax.ShapeDtypeStruct((_TOPK, 5), jnp.float32),
            jax.ShapeDtypeStruct((_TOPK, 1), jnp.int32),
            jax.ShapeDtypeStruct((_TOPK, 1), jnp.int32),
        ),
        scratch_shapes=[pltpu.VMEM((_M, _M), jnp.float32)],
    )(boxes, boxes.T, vals.reshape(1, _M), vals.reshape(_M, 1),
      idx.reshape(1, _M), idx.reshape(_M, 1))
    return out, cls.reshape(-1), ind.reshape(-1)
